# Initial kernel scaffold; baseline (speedup 1.0000x reference)
import functools

import jax
import jax.numpy as jnp
from jax.experimental import pallas as pl
from jax.experimental.pallas import tpu as pltpu

DIM = 768
HEADS = 12
HD = DIM // HEADS
NUM_EXPERTS = 8
HIDDEN = DIM * 4
S = 2048
SBLK = 512


def _rmsnorm(x, dim):
    n = jnp.sqrt(jnp.sum(x * x, axis=-1, keepdims=True))
    return x / jnp.maximum(n, 1e-12) * (dim ** 0.5)


def _attn_kernel(x_ref, wq_ref, wk_ref, wv_ref, ao_ref, hn_out_ref,
                 hn_ref, k_ref, v_ref):
    h = pl.program_id(0)

    @pl.when(h == 0)
    def _init():
        hn = _rmsnorm(x_ref[...], DIM)
        hn_ref[...] = hn
        hn_out_ref[...] = hn
        k_ref[...] = jnp.dot(hn, wk_ref[...], preferred_element_type=jnp.float32)
        v_ref[...] = jnp.dot(hn, wv_ref[...], preferred_element_type=jnp.float32)

    qh = jnp.dot(hn_ref[...], wq_ref[...], preferred_element_type=jnp.float32)
    scores = jax.lax.dot_general(
        qh, k_ref[...], (((1,), (1,)), ((), ())),
        preferred_element_type=jnp.float32) * (1.0 / (HD ** 0.5))
    m = jnp.max(scores, axis=-1, keepdims=True)
    e = jnp.exp(scores - m)
    attnw = e / jnp.sum(e, axis=-1, keepdims=True)
    ao_ref[...] = jnp.dot(attnw, v_ref[...], preferred_element_type=jnp.float32)


def _post_kernel(ao_ref, hn_ref, wo_ref, wg_ref, h2_ref, g_ref):
    t = jnp.dot(ao_ref[...], wo_ref[...], preferred_element_type=jnp.float32)
    t = t + hn_ref[...]
    h2 = _rmsnorm(t, DIM)
    h2_ref[...] = h2
    logits = jnp.dot(h2, wg_ref[...], preferred_element_type=jnp.float32)
    lm = jnp.max(logits, axis=-1, keepdims=True)
    ex = jnp.exp(logits - lm)
    gates = ex / jnp.sum(ex, axis=-1, keepdims=True)
    m1 = jnp.max(gates, axis=-1, keepdims=True)
    lanes = jax.lax.broadcasted_iota(jnp.int32, gates.shape, 1)
    e1 = jnp.argmax(gates, axis=-1)[:, None]
    g2 = jnp.where(lanes == e1, -jnp.inf, gates)
    m2 = jnp.max(g2, axis=-1, keepdims=True)
    e2 = jnp.argmax(g2, axis=-1)[:, None]
    keep = (lanes == e1) | (lanes == e2)
    g = jnp.where(keep, gates, 0.0)
    g_ref[...] = g / (m1 + m2)


def _moe_kernel(h2_ref, skip_ref, gt_ref, w1_ref, b1_ref, w2_ref, b2_ref,
                out_ref):
    e = pl.program_id(1)

    @pl.when(e == 0)
    def _init():
        out_ref[...] = skip_ref[...]

    hid = jnp.dot(h2_ref[...], w1_ref[0], preferred_element_type=jnp.float32)
    hid = jax.nn.gelu(hid + b1_ref[0][None, :])
    eo = jnp.dot(hid, w2_ref[0], preferred_element_type=jnp.float32)
    eo = eo + b2_ref[0][None, :]
    out_ref[...] += gt_ref[0] * eo


def kernel(x, Wq, Wk, Wv, Wo, Wg, W1, b1, W2, b2):
    xs = x.reshape(S, DIM)

    ao, hn = pl.pallas_call(
        _attn_kernel,
        grid=(HEADS,),
        in_specs=[
            pl.BlockSpec((S, DIM), lambda h: (0, 0)),
            pl.BlockSpec((DIM, HD), lambda h: (0, h)),
            pl.BlockSpec((DIM, HD), lambda h: (0, 0)),
            pl.BlockSpec((DIM, HD), lambda h: (0, 0)),
        ],
        out_specs=[
            pl.BlockSpec((S, HD), lambda h: (0, h)),
            pl.BlockSpec((S, DIM), lambda h: (0, 0)),
        ],
        out_shape=[
            jax.ShapeDtypeStruct((S, DIM), jnp.float32),
            jax.ShapeDtypeStruct((S, DIM), jnp.float32),
        ],
        scratch_shapes=[
            pltpu.VMEM((S, DIM), jnp.float32),
            pltpu.VMEM((S, HD), jnp.float32),
            pltpu.VMEM((S, HD), jnp.float32),
        ],
    )(xs, Wq, Wk, Wv)

    h2, g = pl.pallas_call(
        _post_kernel,
        grid=(S // SBLK,),
        in_specs=[
            pl.BlockSpec((SBLK, DIM), lambda i: (i, 0)),
            pl.BlockSpec((SBLK, DIM), lambda i: (i, 0)),
            pl.BlockSpec((DIM, DIM), lambda i: (0, 0)),
            pl.BlockSpec((DIM, NUM_EXPERTS), lambda i: (0, 0)),
        ],
        out_specs=[
            pl.BlockSpec((SBLK, DIM), lambda i: (i, 0)),
            pl.BlockSpec((SBLK, NUM_EXPERTS), lambda i: (i, 0)),
        ],
        out_shape=[
            jax.ShapeDtypeStruct((S, DIM), jnp.float32),
            jax.ShapeDtypeStruct((S, NUM_EXPERTS), jnp.float32),
        ],
    )(ao, hn, Wo, Wg)

    gt = g.T.reshape(NUM_EXPERTS, S, 1)

    out = pl.pallas_call(
        _moe_kernel,
        grid=(S // SBLK, NUM_EXPERTS),
        in_specs=[
            pl.BlockSpec((SBLK, DIM), lambda s, e: (s, 0)),
            pl.BlockSpec((SBLK, DIM), lambda s, e: (s, 0)),
            pl.BlockSpec((1, SBLK, 1), lambda s, e: (e, s, 0)),
            pl.BlockSpec((1, DIM, HIDDEN), lambda s, e: (e, 0, 0)),
            pl.BlockSpec((1, HIDDEN), lambda s, e: (e, 0)),
            pl.BlockSpec((1, HIDDEN, DIM), lambda s, e: (e, 0, 0)),
            pl.BlockSpec((1, DIM), lambda s, e: (e, 0)),
        ],
        out_specs=pl.BlockSpec((SBLK, DIM), lambda s, e: (s, 0)),
        out_shape=jax.ShapeDtypeStruct((S, DIM), jnp.float32),
    )(h2, xs, gt, W1, b1, W2, b2)

    return out.reshape(1, S, DIM)


# speedup vs baseline: 1.2127x; 1.2127x over previous
import functools

import jax
import jax.numpy as jnp
from jax.experimental import pallas as pl
from jax.experimental.pallas import tpu as pltpu

DIM = 768
HEADS = 12
HD = DIM // HEADS
NUM_EXPERTS = 8
HIDDEN = DIM * 4
S = 2048
SBLK = 512


def _rmsnorm(x, dim):
    n = jnp.sqrt(jnp.sum(x * x, axis=-1, keepdims=True))
    return x / jnp.maximum(n, 1e-12) * (dim ** 0.5)


def _attn_kernel(x_ref, wq_ref, wk_ref, wv_ref, ao_ref, hn_out_ref,
                 hn_ref, k_ref, v_ref):
    h = pl.program_id(0)

    @pl.when(h == 0)
    def _init():
        hn = _rmsnorm(x_ref[...], DIM)
        hn_ref[...] = hn
        hn_out_ref[...] = hn
        k_ref[...] = jnp.dot(hn, wk_ref[...], preferred_element_type=jnp.float32)
        v_ref[...] = jnp.dot(hn, wv_ref[...], preferred_element_type=jnp.float32)

    qh = jnp.dot(hn_ref[...], wq_ref[0], preferred_element_type=jnp.float32)
    scores = jax.lax.dot_general(
        qh, k_ref[...], (((1,), (1,)), ((), ())),
        preferred_element_type=jnp.float32) * (1.0 / (HD ** 0.5))
    m = jnp.max(scores, axis=-1, keepdims=True)
    e = jnp.exp(scores - m)
    attnw = e / jnp.sum(e, axis=-1, keepdims=True)
    ao_ref[0] = jnp.dot(attnw, v_ref[...], preferred_element_type=jnp.float32)


def _post_kernel(ao_ref, hn_ref, wo_ref, wg_ref, h2_ref, g_ref):
    t = hn_ref[...]
    for h in range(HEADS):
        t = t + jnp.dot(ao_ref[h], wo_ref[h],
                        preferred_element_type=jnp.float32)
    h2 = _rmsnorm(t, DIM)
    h2_ref[...] = h2
    logits = jnp.dot(h2, wg_ref[...], preferred_element_type=jnp.float32)
    lm = jnp.max(logits, axis=-1, keepdims=True)
    ex = jnp.exp(logits - lm)
    gates = ex / jnp.sum(ex, axis=-1, keepdims=True)
    m1 = jnp.max(gates, axis=-1, keepdims=True)
    lanes = jax.lax.broadcasted_iota(jnp.int32, gates.shape, 1)
    e1 = jnp.argmax(gates, axis=-1)[:, None]
    g2 = jnp.where(lanes == e1, -jnp.inf, gates)
    m2 = jnp.max(g2, axis=-1, keepdims=True)
    e2 = jnp.argmax(g2, axis=-1)[:, None]
    keep = (lanes == e1) | (lanes == e2)
    g = jnp.where(keep, gates, 0.0)
    g_ref[...] = g / (m1 + m2)


def _moe_kernel(h2_ref, skip_ref, gt_ref, w1_ref, b1_ref, w2_ref, b2_ref,
                out_ref):
    e = pl.program_id(1)

    @pl.when(e == 0)
    def _init():
        out_ref[...] = skip_ref[...]

    hid = jnp.dot(h2_ref[...], w1_ref[0], preferred_element_type=jnp.float32)
    hid = jax.nn.gelu(hid + b1_ref[0])
    eo = jnp.dot(hid, w2_ref[0], preferred_element_type=jnp.float32)
    eo = eo + b2_ref[0]
    out_ref[...] += gt_ref[0] * eo


def kernel(x, Wq, Wk, Wv, Wo, Wg, W1, b1, W2, b2):
    xs = x.reshape(S, DIM)
    wq3 = Wq.reshape(DIM, HEADS, HD).transpose(1, 0, 2)
    wo3 = Wo.reshape(HEADS, HD, DIM)

    ao, hn = pl.pallas_call(
        _attn_kernel,
        grid=(HEADS,),
        in_specs=[
            pl.BlockSpec((S, DIM), lambda h: (0, 0)),
            pl.BlockSpec((1, DIM, HD), lambda h: (h, 0, 0)),
            pl.BlockSpec((DIM, HD), lambda h: (0, 0)),
            pl.BlockSpec((DIM, HD), lambda h: (0, 0)),
        ],
        out_specs=[
            pl.BlockSpec((1, S, HD), lambda h: (h, 0, 0)),
            pl.BlockSpec((S, DIM), lambda h: (0, 0)),
        ],
        out_shape=[
            jax.ShapeDtypeStruct((HEADS, S, HD), jnp.float32),
            jax.ShapeDtypeStruct((S, DIM), jnp.float32),
        ],
        scratch_shapes=[
            pltpu.VMEM((S, DIM), jnp.float32),
            pltpu.VMEM((S, HD), jnp.float32),
            pltpu.VMEM((S, HD), jnp.float32),
        ],
    )(xs, wq3, Wk, Wv)

    h2, g = pl.pallas_call(
        _post_kernel,
        grid=(S // SBLK,),
        in_specs=[
            pl.BlockSpec((HEADS, SBLK, HD), lambda i: (0, i, 0)),
            pl.BlockSpec((SBLK, DIM), lambda i: (i, 0)),
            pl.BlockSpec((HEADS, HD, DIM), lambda i: (0, 0, 0)),
            pl.BlockSpec((DIM, NUM_EXPERTS), lambda i: (0, 0)),
        ],
        out_specs=[
            pl.BlockSpec((SBLK, DIM), lambda i: (i, 0)),
            pl.BlockSpec((SBLK, NUM_EXPERTS), lambda i: (i, 0)),
        ],
        out_shape=[
            jax.ShapeDtypeStruct((S, DIM), jnp.float32),
            jax.ShapeDtypeStruct((S, NUM_EXPERTS), jnp.float32),
        ],
    )(ao, hn, wo3, Wg)

    gt = g.T.reshape(NUM_EXPERTS, S, 1)

    out = pl.pallas_call(
        _moe_kernel,
        grid=(S // SBLK, NUM_EXPERTS),
        in_specs=[
            pl.BlockSpec((SBLK, DIM), lambda s, e: (s, 0)),
            pl.BlockSpec((SBLK, DIM), lambda s, e: (s, 0)),
            pl.BlockSpec((1, SBLK, 1), lambda s, e: (e, s, 0)),
            pl.BlockSpec((1, DIM, HIDDEN), lambda s, e: (e, 0, 0)),
            pl.BlockSpec((1, 1, HIDDEN), lambda s, e: (e, 0, 0)),
            pl.BlockSpec((1, HIDDEN, DIM), lambda s, e: (e, 0, 0)),
            pl.BlockSpec((1, 1, DIM), lambda s, e: (e, 0, 0)),
        ],
        out_specs=pl.BlockSpec((SBLK, DIM), lambda s, e: (s, 0)),
        out_shape=jax.ShapeDtypeStruct((S, DIM), jnp.float32),
    )(h2, xs, gt, W1, b1.reshape(NUM_EXPERTS, 1, HIDDEN), W2,
      b2.reshape(NUM_EXPERTS, 1, DIM))

    return out.reshape(1, S, DIM)


# speedup vs baseline: 1.3541x; 1.1166x over previous
import functools

import jax
import jax.numpy as jnp
from jax import lax
from jax.experimental import pallas as pl
from jax.experimental.pallas import tpu as pltpu
from jax.experimental.pallas import tpu_sc as plsc

DIM = 768
HEADS = 12
HD = DIM // HEADS
NUM_EXPERTS = 8
HIDDEN = DIM * 4
S = 2048
SBLK = 512

TOPK = 2
NSLOT = TOPK * S
BS = 256
NBLK = NSLOT // BS + NUM_EXPERTS
NROWS = NBLK * BS

NW = 32
RPW = NROWS // NW
GCH = RPW // 2
TPW = S // NW


def _rmsnorm(x, dim):
    n = jnp.sqrt(jnp.sum(x * x, axis=-1, keepdims=True))
    return x / jnp.maximum(n, 1e-12) * (dim ** 0.5)


def _attn_kernel(x_ref, wq_ref, wk_ref, wv_ref, ao_ref, hn_out_ref,
                 hn_ref, k_ref, v_ref):
    h = pl.program_id(0)

    @pl.when(h == 0)
    def _init():
        hn = _rmsnorm(x_ref[...], DIM)
        hn_ref[...] = hn
        hn_out_ref[...] = hn
        k_ref[...] = jnp.dot(hn, wk_ref[...], preferred_element_type=jnp.float32)
        v_ref[...] = jnp.dot(hn, wv_ref[...], preferred_element_type=jnp.float32)

    qh = jnp.dot(hn_ref[...], wq_ref[0], preferred_element_type=jnp.float32)
    scores = lax.dot_general(
        qh, k_ref[...], (((1,), (1,)), ((), ())),
        preferred_element_type=jnp.float32) * (1.0 / (HD ** 0.5))
    m = jnp.max(scores, axis=-1, keepdims=True)
    e = jnp.exp(scores - m)
    attnw = e / jnp.sum(e, axis=-1, keepdims=True)
    ao_ref[0] = jnp.dot(attnw, v_ref[...], preferred_element_type=jnp.float32)


def _post_kernel(ao_ref, hn_ref, wo_ref, wg_ref,
                 h2_ref, w1_ref, w2_ref, p1_ref, p2_ref, cnt_ref):
    t = hn_ref[...]
    for h in range(HEADS):
        t = t + jnp.dot(ao_ref[h], wo_ref[h],
                        preferred_element_type=jnp.float32)
    h2 = _rmsnorm(t, DIM)
    h2_ref[...] = h2

    logits = jnp.dot(h2, wg_ref[...], preferred_element_type=jnp.float32)
    lm = jnp.max(logits, axis=-1, keepdims=True)
    ex = jnp.exp(logits - lm)
    gates = ex / jnp.sum(ex, axis=-1, keepdims=True)
    lanes = lax.broadcasted_iota(jnp.int32, gates.shape, 1)
    m1 = jnp.max(gates, axis=-1, keepdims=True)
    e1 = jnp.argmax(gates, axis=-1)[:, None]
    gm = jnp.where(lanes == e1, -jnp.inf, gates)
    m2 = jnp.max(gm, axis=-1, keepdims=True)
    e2 = jnp.argmax(gm, axis=-1)[:, None]
    denom = m1 + m2
    w1_ref[...] = m1 / denom
    w2_ref[...] = m2 / denom

    oh1 = (lanes == e1).astype(jnp.float32)
    oh2 = (lanes == e2).astype(jnp.float32)
    r = lax.broadcasted_iota(jnp.int32, (S, S), 0)
    c = lax.broadcasted_iota(jnp.int32, (S, S), 1)
    ltri = (c <= r).astype(jnp.float32)
    c1 = jnp.dot(ltri, oh1, preferred_element_type=jnp.float32)
    c2 = jnp.dot(ltri, oh2, preferred_element_type=jnp.float32)
    cnt1 = jnp.sum(oh1, axis=0, keepdims=True)
    cnt2 = jnp.sum(oh2, axis=0, keepdims=True)
    counts = cnt1 + cnt2
    nblk = jnp.floor((counts + (BS - 1)) * (1.0 / BS))
    r8 = lax.broadcasted_iota(jnp.int32, (NUM_EXPERTS, NUM_EXPERTS), 0)
    c8 = lax.broadcasted_iota(jnp.int32, (NUM_EXPERTS, NUM_EXPERTS), 1)
    strict = (r8 < c8).astype(jnp.float32)
    rowbase = jnp.dot(nblk, strict,
                      preferred_element_type=jnp.float32) * BS

    def pick(mat, oh):
        return jnp.sum(mat * oh, axis=1, keepdims=True)

    p1 = pick(rowbase + c1 - 1.0, oh1)
    p2 = pick(rowbase + cnt1 + c2 - 1.0, oh2)
    p1_ref[...] = p1.astype(jnp.int32)
    p2_ref[...] = p2.astype(jnp.int32)
    cnt_ref[...] = counts.astype(jnp.int32)


def _sc_dispatch(p12_hbm, h2_hbm, xs_hbm, idx2, rows_v, sem):
    wid = lax.axis_index("s") * 2 + lax.axis_index("c")
    nsl = NSLOT // NW
    slo = wid * nsl
    tlo = lax.rem(slo, S)
    pltpu.sync_copy(p12_hbm.at[pl.ds(slo, nsl)], idx2.at[0])
    pltpu.sync_copy(h2_hbm.at[pl.ds(tlo, nsl)], rows_v)
    pltpu.async_copy(rows_v, xs_hbm.at[idx2.at[0]], sem).wait()


def _ffn_kernel(eob_ref, xs_ref, w1_ref, b1_ref, w2_ref, b2_ref, y_ref):
    hid = jnp.dot(xs_ref[...], w1_ref[0], preferred_element_type=jnp.float32)
    hid = jax.nn.gelu(hid + b1_ref[0])
    y_ref[...] = jnp.dot(hid, w2_ref[0],
                         preferred_element_type=jnp.float32) + b2_ref[0]


def _sc_combine(p1_hbm, p2_hbm, ypad_hbm, y1_hbm, y2_hbm,
                idx_v, rows_v, sem):
    wid = lax.axis_index("s") * 2 + lax.axis_index("c")
    lo = wid * TPW
    pltpu.sync_copy(p1_hbm.at[pl.ds(lo, TPW)], idx_v)
    pltpu.async_copy(ypad_hbm.at[idx_v], rows_v, sem).wait()
    pltpu.sync_copy(rows_v, y1_hbm.at[pl.ds(lo, TPW)])
    pltpu.sync_copy(p2_hbm.at[pl.ds(lo, TPW)], idx_v)
    pltpu.async_copy(ypad_hbm.at[idx_v], rows_v, sem).wait()
    pltpu.sync_copy(rows_v, y2_hbm.at[pl.ds(lo, TPW)])


def _combine_kernel(skip_ref, y1_ref, y2_ref, w1_ref, w2_ref, out_ref):
    out_ref[...] = (skip_ref[...] + w1_ref[...] * y1_ref[...]
                    + w2_ref[...] * y2_ref[...])


def kernel(x, Wq, Wk, Wv, Wo, Wg, W1, b1, W2, b2):
    xs = x.reshape(S, DIM)
    wq3 = Wq.reshape(DIM, HEADS, HD).transpose(1, 0, 2)
    wo3 = Wo.reshape(HEADS, HD, DIM)

    ao, hn = pl.pallas_call(
        _attn_kernel,
        grid=(HEADS,),
        in_specs=[
            pl.BlockSpec((S, DIM), lambda h: (0, 0)),
            pl.BlockSpec((1, DIM, HD), lambda h: (h, 0, 0)),
            pl.BlockSpec((DIM, HD), lambda h: (0, 0)),
            pl.BlockSpec((DIM, HD), lambda h: (0, 0)),
        ],
        out_specs=[
            pl.BlockSpec((1, S, HD), lambda h: (h, 0, 0)),
            pl.BlockSpec((S, DIM), lambda h: (0, 0)),
        ],
        out_shape=[
            jax.ShapeDtypeStruct((HEADS, S, HD), jnp.float32),
            jax.ShapeDtypeStruct((S, DIM), jnp.float32),
        ],
        scratch_shapes=[
            pltpu.VMEM((S, DIM), jnp.float32),
            pltpu.VMEM((S, HD), jnp.float32),
            pltpu.VMEM((S, HD), jnp.float32),
        ],
    )(xs, wq3, Wk, Wv)

    h2, w1g, w2g, p1, p2, counts = pl.pallas_call(
        _post_kernel,
        grid=(1,),
        in_specs=[
            pl.BlockSpec((HEADS, S, HD), lambda i: (0, 0, 0)),
            pl.BlockSpec((S, DIM), lambda i: (0, 0)),
            pl.BlockSpec((HEADS, HD, DIM), lambda i: (0, 0, 0)),
            pl.BlockSpec((DIM, NUM_EXPERTS), lambda i: (0, 0)),
        ],
        out_specs=[
            pl.BlockSpec((S, DIM), lambda i: (0, 0)),
            pl.BlockSpec((S, 1), lambda i: (0, 0)),
            pl.BlockSpec((S, 1), lambda i: (0, 0)),
            pl.BlockSpec((S, 1), lambda i: (0, 0)),
            pl.BlockSpec((S, 1), lambda i: (0, 0)),
            pl.BlockSpec((1, NUM_EXPERTS), lambda i: (0, 0)),
        ],
        out_shape=[
            jax.ShapeDtypeStruct((S, DIM), jnp.float32),
            jax.ShapeDtypeStruct((S, 1), jnp.float32),
            jax.ShapeDtypeStruct((S, 1), jnp.float32),
            jax.ShapeDtypeStruct((S, 1), jnp.int32),
            jax.ShapeDtypeStruct((S, 1), jnp.int32),
            jax.ShapeDtypeStruct((1, NUM_EXPERTS), jnp.int32),
        ],
    )(ao, hn, wo3, Wg)

    p1f = p1.reshape(S)
    p2f = p2.reshape(S)
    p12 = jnp.concatenate([p1f, p2f])

    mesh = plsc.VectorSubcoreMesh(core_axis_name="c", subcore_axis_name="s",
                                  num_cores=2, num_subcores=16)
    xsrows = pl.kernel(
        _sc_dispatch,
        out_type=jax.ShapeDtypeStruct((NROWS, DIM), jnp.float32),
        mesh=mesh,
        scratch_types=[
            pltpu.VMEM((2, NSLOT // NW), jnp.int32),
            pltpu.VMEM((NSLOT // NW, DIM), jnp.float32),
            pltpu.SemaphoreType.DMA,
        ],
    )(p12, h2)

    nblk = (counts[0] + (BS - 1)) // BS
    eob = jnp.repeat(jnp.arange(NUM_EXPERTS, dtype=jnp.int32), nblk,
                     total_repeat_length=NBLK)

    ypad = pl.pallas_call(
        _ffn_kernel,
        grid_spec=pltpu.PrefetchScalarGridSpec(
            num_scalar_prefetch=1,
            grid=(NBLK,),
            in_specs=[
                pl.BlockSpec((BS, DIM), lambda b, eob: (b, 0)),
                pl.BlockSpec((1, DIM, HIDDEN), lambda b, eob: (eob[b], 0, 0)),
                pl.BlockSpec((1, 1, HIDDEN), lambda b, eob: (eob[b], 0, 0)),
                pl.BlockSpec((1, HIDDEN, DIM), lambda b, eob: (eob[b], 0, 0)),
                pl.BlockSpec((1, 1, DIM), lambda b, eob: (eob[b], 0, 0)),
            ],
            out_specs=pl.BlockSpec((BS, DIM), lambda b, eob: (b, 0)),
        ),
        out_shape=jax.ShapeDtypeStruct((NROWS, DIM), jnp.float32),
    )(eob, xsrows, W1, b1.reshape(NUM_EXPERTS, 1, HIDDEN), W2,
      b2.reshape(NUM_EXPERTS, 1, DIM))

    y1, y2 = pl.kernel(
        _sc_combine,
        out_type=[
            jax.ShapeDtypeStruct((S, DIM), jnp.float32),
            jax.ShapeDtypeStruct((S, DIM), jnp.float32),
        ],
        mesh=mesh,
        scratch_types=[
            pltpu.VMEM((TPW,), jnp.int32),
            pltpu.VMEM((TPW, DIM), jnp.float32),
            pltpu.SemaphoreType.DMA,
        ],
    )(p1f, p2f, ypad)

    out = pl.pallas_call(
        _combine_kernel,
        grid=(S // SBLK,),
        in_specs=[
            pl.BlockSpec((SBLK, DIM), lambda i: (i, 0)),
            pl.BlockSpec((SBLK, DIM), lambda i: (i, 0)),
            pl.BlockSpec((SBLK, DIM), lambda i: (i, 0)),
            pl.BlockSpec((SBLK, 1), lambda i: (i, 0)),
            pl.BlockSpec((SBLK, 1), lambda i: (i, 0)),
        ],
        out_specs=pl.BlockSpec((SBLK, DIM), lambda i: (i, 0)),
        out_shape=jax.ShapeDtypeStruct((S, DIM), jnp.float32),
    )(xs, y1, y2, w1g, w2g)

    return out.reshape(1, S, DIM)
